# Initial kernel scaffold; baseline (speedup 1.0000x reference)
import jax
import jax.numpy as jnp
from jax.experimental import pallas as pl


def kernel(x1_features, x1_pos, x1_batch, x2_features, x2_pos, x2_batch, params):
    raise NotImplementedError("write your pallas kernel here")


# speedup vs baseline: 6.4709x; 6.4709x over previous
import functools

import jax
import jax.numpy as jnp
from jax import lax
from jax.experimental import pallas as pl
from jax.experimental.pallas import tpu as pltpu
from jax.experimental.pallas import tpu_sc as plsc

@functools.lru_cache(maxsize=None)
def _knn2_build(nq, nr, tq):
    grid = nq // tq

    def bf(x):
        return x.astype(jnp.bfloat16).astype(jnp.float32)

    def body(qx, qy, qz, rx, ry, rz, oi0, oi1, od0, od1):
        qxv, qyv, qzv = qx[...], qy[...], qz[...]
        rxv, ryv, rzv = rx[...], ry[...], rz[...]
        qq = (qxv * qxv + qzv * qzv) + qyv * qyv
        rr = (rxv * rxv + rzv * rzv) + ryv * ryv
        qr = (bf(qxv) * bf(rxv) + bf(qyv) * bf(ryv)) + bf(qzv) * bf(rzv)
        d = (qq + rr) - 2.0 * qr
        lane = lax.broadcasted_iota(jnp.int32, (tq, nr), 1)
        m1 = jnp.min(d, axis=1, keepdims=True)
        i1 = jnp.min(jnp.where(d == m1, lane, (2**30)), axis=1, keepdims=True)
        dm = jnp.where(lane == i1, jnp.inf, d)
        m2 = jnp.min(dm, axis=1, keepdims=True)
        i2 = jnp.min(jnp.where((d == m2) & (lane != i1), lane, (2**30)),
                     axis=1, keepdims=True)
        oi0[...] = i1
        oi1[...] = i2
        od0[...] = m1
        od1[...] = m2

    qspec = pl.BlockSpec((tq, 1), lambda i: (i, 0))
    rspec = pl.BlockSpec((1, nr), lambda i: (0, 0))
    ospec = pl.BlockSpec((tq, 1), lambda i: (i, 0))
    return pl.pallas_call(
        body,
        grid=(grid,),
        in_specs=[qspec] * 3 + [rspec] * 3,
        out_specs=[ospec] * 4,
        out_shape=[
            jax.ShapeDtypeStruct((nq, 1), jnp.int32),
            jax.ShapeDtypeStruct((nq, 1), jnp.int32),
            jax.ShapeDtypeStruct((nq, 1), jnp.float32),
            jax.ShapeDtypeStruct((nq, 1), jnp.float32),
        ],
    )


def _knn2(qc, rc, tq=128):
    nq = qc[0].shape[0]
    nr = rc[0].shape[1]
    return _knn2_build(nq, nr, tq)(*qc, *rc)


@functools.lru_cache(maxsize=None)
def _fps_build(n, m):
    rows = n // 128

    def body(xs, ys, zs, oidx, ox, oy, oz, dist):
        xsv, ysv, zsv = xs[...], ys[...], zs[...]
        rowi = lax.broadcasted_iota(jnp.int32, (rows, 128), 0)
        coli = lax.broadcasted_iota(jnp.int32, (rows, 128), 1)
        flat = rowi * 128 + coli
        dist[...] = jnp.full((rows, 128), jnp.inf, jnp.float32)

        def step(t, last):
            sel = flat == last
            xl = jnp.max(jnp.where(sel, xsv, -1.0))
            yl = jnp.max(jnp.where(sel, ysv, -1.0))
            zl = jnp.max(jnp.where(sel, zsv, -1.0))
            oidx[pl.ds(t, 1), :] = jnp.reshape(last, (1, 1))
            ox[pl.ds(t, 1), :] = jnp.reshape(xl, (1, 1))
            oy[pl.ds(t, 1), :] = jnp.reshape(yl, (1, 1))
            oz[pl.ds(t, 1), :] = jnp.reshape(zl, (1, 1))
            dx = xsv - xl
            dy = ysv - yl
            dz = zsv - zl
            d = (dx * dx + dz * dz) + dy * dy
            nd = jnp.minimum(dist[...], d)
            dist[...] = nd
            mv = jnp.max(nd)
            return jnp.min(jnp.where(nd == mv, flat, (2**30)))

        lax.fori_loop(0, m, step, jnp.int32(0))

    return pl.pallas_call(
        body,
        out_shape=[
            jax.ShapeDtypeStruct((m, 1), jnp.int32),
            jax.ShapeDtypeStruct((m, 1), jnp.float32),
            jax.ShapeDtypeStruct((m, 1), jnp.float32),
            jax.ShapeDtypeStruct((m, 1), jnp.float32),
        ],
        scratch_shapes=[pltpu.VMEM((rows, 128), jnp.float32)],
    )


def _fps(xc, yc, zc, m):
    n = xc.shape[0]
    xs = xc.reshape(n // 128, 128)
    ys = yc.reshape(n // 128, 128)
    zs = zc.reshape(n // 128, 128)
    return _fps_build(n, m)(xs, ys, zs)


def _dotbf(a, w):
    return jnp.dot(a.astype(jnp.bfloat16), w.astype(jnp.bfloat16),
                   preferred_element_type=jnp.float32)


@functools.lru_cache(maxsize=None)
def _prep_fe_build(n, c):
    npad = 125

    def body(f2, f1, wa, wb, b1, x2, y2, z2, og, oh):
        pad = jnp.broadcast_to(x2[...] * 0.0, (n, npad))
        og[...] = jnp.concatenate(
            [_dotbf(f2[...], wa[...]), x2[...], y2[...], z2[...], pad],
            axis=1)
        oh[...] = _dotbf(f1[...], wb[...]) + b1[...]

    return pl.pallas_call(
        body,
        out_shape=[
            jax.ShapeDtypeStruct((n, c + 3 + npad), jnp.float32),
            jax.ShapeDtypeStruct((n, c), jnp.float32),
        ],
    )


@functools.lru_cache(maxsize=None)
def _prep_sc_build(n, c):
    npad = 125

    def body(f, wa, xc, yc, zc, og):
        pad = jnp.broadcast_to(xc[...] * 0.0, (n, npad))
        og[...] = jnp.concatenate(
            [_dotbf(f[...], wa[...]), xc[...], yc[...], zc[...], pad],
            axis=1)

    return pl.pallas_call(
        body,
        out_shape=jax.ShapeDtypeStruct((n, c + 3 + npad), jnp.float32),
    )


@functools.lru_cache(maxsize=None)
def _gather_build(v, d, b):
    nw = 32
    b_per_w = b // nw
    chunk = min(b_per_w, 128)
    n_chunks = b_per_w // chunk
    mesh = plsc.VectorSubcoreMesh(core_axis_name="c", subcore_axis_name="s")

    @functools.partial(
        pl.kernel,
        mesh=mesh,
        out_type=jax.ShapeDtypeStruct((b, d), jnp.float32),
        scratch_types=[
            pltpu.VMEM((n_chunks, chunk), jnp.int32),
            pltpu.VMEM((chunk, d), jnp.float32),
            pltpu.SemaphoreType.DMA,
        ],
    )
    def k(idx_hbm, table_hbm, out_hbm, idx_v, rows_v, sem):
        wid = lax.axis_index("s") * 2 + lax.axis_index("c")
        pltpu.sync_copy(idx_hbm.at[pl.ds(wid * n_chunks, n_chunks)], idx_v)
        for j in range(n_chunks):
            pltpu.async_copy(table_hbm.at[idx_v.at[j]], rows_v, sem).wait()
            pltpu.sync_copy(
                rows_v, out_hbm.at[pl.ds(wid * b_per_w + j * chunk, chunk)])

    return k, chunk


def _gather(idx, table):
    v, d = table.shape
    b = idx.shape[0]
    k, chunk = _gather_build(v, d, b)
    return k(idx.reshape(b // chunk, chunk), table)


@functools.lru_cache(maxsize=None)
def _mlp_build(nq, c1, c2, c3, r2, h_rows):
    inv_cnt = 1.0 / (2.0 * nq)

    def bf(x):
        return x.astype(jnp.bfloat16).astype(jnp.float32)

    def bn(a0, a1, g, be):
        mu = (jnp.sum(a0, axis=0, keepdims=True)
              + jnp.sum(a1, axis=0, keepdims=True)) * inv_cnt
        d0 = a0 - mu
        d1 = a1 - mu
        var = (jnp.sum(d0 * d0, axis=0, keepdims=True)
               + jnp.sum(d1 * d1, axis=0, keepdims=True)) * inv_cnt
        s = jnp.sqrt(var + 1e-5)
        return d0 / s * g + be, d1 / s * g + be

    def body(z0, z1, h, pc, w0, w1, w2p, w2, b2, w3, b3,
             g1, be1, g2, be2, g3, be3, out):
        w0v, w1v, w2v = bf(w0[...]), bf(w1[...]), bf(w2p[...])
        pcv = pc[...]
        qxv = pcv[:, 0:1]
        qyv = pcv[:, 1:2]
        qzv = pcv[:, 2:3]
        dv0 = pcv[:, 3:4]
        dv1 = pcv[:, 4:5]
        hv = h[...]

        def layer1(z):
            zv = z[...]
            rx = bf(zv[:, c1:c1 + 1] - qxv)
            ry = bf(zv[:, c1 + 1:c1 + 2] - qyv)
            rz = bf(zv[:, c1 + 2:c1 + 3] - qzv)
            return (zv[:, 0:c1] + hv) + ((rx * w0v + ry * w1v) + rz * w2v)

        a0 = layer1(z0)
        a1 = layer1(z1)
        a0, a1 = bn(a0, a1, g1[...], be1[...])
        a0 = jnp.maximum(a0, 0.0)
        a1 = jnp.maximum(a1, 0.0)
        t0 = _dotbf(a0, w2[...]) + b2[...]
        t1 = _dotbf(a1, w2[...]) + b2[...]
        a0, a1 = bn(t0, t1, g2[...], be2[...])
        a0 = jnp.maximum(a0, 0.0)
        a1 = jnp.maximum(a1, 0.0)
        t0 = _dotbf(a0, w3[...]) + b3[...]
        t1 = _dotbf(a1, w3[...]) + b3[...]
        a0, a1 = bn(t0, t1, g3[...], be3[...])
        a0 = jnp.maximum(a0, 0.0)
        a1 = jnp.maximum(a1, 0.0)
        mask0 = dv0 <= r2
        mask1 = dv1 <= r2
        mx = jnp.maximum(jnp.where(mask0, a0, -1e30),
                         jnp.where(mask1, a1, -1e30))
        out[...] = jnp.where(mask0 | mask1, mx, 0.0)

    return pl.pallas_call(
        body,
        out_shape=jax.ShapeDtypeStruct((nq, c3), jnp.float32),
    )


def _mlp(z0, z1, h, qc, wc_rows, dv, lyr2, lyr3, gbe1, r2):
    nq = z0.shape[0]
    c1 = lyr2[0].shape[0]
    w2, b2, g2, be2 = lyr2
    w3, b3, g3, be3 = lyr3
    g1, be1 = gbe1
    pc = jnp.concatenate(list(qc) + [dv[0], dv[1]], axis=1)
    return _mlp_build(nq, c1, w2.shape[1], w3.shape[1], r2, h.shape[0])(
        z0, z1, h, pc, *wc_rows, w2, _r(b2), w3, _r(b3),
        _r(g1), _r(be1), _r(g2), _r(be2), _r(g3), _r(be3))


def _cols(p):
    return p[:, 0:1], p[:, 1:2], p[:, 2:3]


def _r(x):
    return x.reshape(1, -1)


def kernel(x1_features, x1_pos, x1_batch, x2_features, x2_pos, x2_batch,
           params):
    n = x1_features.shape[0]
    p1 = x1_pos
    x1c, y1c, z1c = _cols(p1)
    x2c, y2c, z2c = _cols(x2_pos)

    (w1, b1, g1, be1), l2, l3 = params['fe']
    wa, wb, wc = w1[0:128], w1[128:256], w1[256:259]
    wc_rows = (_r(wc[0]), _r(wc[1]), _r(wc[2]))
    i0, i1, d0, d1 = _knn2((x1c, y1c, z1c), (_r(x2c), _r(y2c), _r(z2c)))
    g_tab, h_q = _prep_fe_build(n, 128)(
        x2_features, x1_features, wa, wb, _r(b1), x2c, y2c, z2c)
    z = _gather(jnp.concatenate([i0.reshape(-1), i1.reshape(-1)]), g_tab)
    fe1 = _mlp(z[:n], z[n:], h_q, (x1c, y1c, z1c), wc_rows, (d0, d1),
               l2, l3, (g1, be1), 25.0)

    (w1, b1, g1, be1), l2, l3 = params['sc1']
    wa, wc = w1[0:128], w1[128:131]
    wc_rows = (_r(wc[0]), _r(wc[1]), _r(wc[2]))
    m1 = 2048
    _, cx, cy, cz = _fps(x1c, y1c, z1c, m1)
    cp = jnp.concatenate([cx, cy, cz], axis=1)
    i0, i1, d0, d1 = _knn2((cx, cy, cz), (_r(x1c), _r(y1c), _r(z1c)))
    g_tab = _prep_sc_build(n, 128)(fe1, wa, x1c, y1c, z1c)
    z = _gather(jnp.concatenate([i0.reshape(-1), i1.reshape(-1)]), g_tab)
    fe2 = _mlp(z[:m1], z[m1:], _r(b1), (cx, cy, cz), wc_rows, (d0, d1),
               l2, l3, (g1, be1), 4.0)

    (w1, b1, g1, be1), l2, l3 = params['sc2']
    wa, wc = w1[0:256], w1[256:259]
    wc_rows = (_r(wc[0]), _r(wc[1]), _r(wc[2]))
    m2 = 512
    _, qx, qy, qz = _fps(cx, cy, cz, m2)
    cp2 = jnp.concatenate([qx, qy, qz], axis=1)
    i0, i1, d0, d1 = _knn2((qx, qy, qz), (_r(cx), _r(cy), _r(cz)))
    g_tab = _prep_sc_build(m1, 256)(fe2, wa, cx, cy, cz)
    z = _gather(jnp.concatenate([i0.reshape(-1), i1.reshape(-1)]), g_tab)
    fe3 = _mlp(z[:m2], z[m2:], _r(b1), (qx, qy, qz), wc_rows, (d0, d1),
               l2, l3, (g1, be1), 16.0)

    return (fe1, p1, fe2, cp, fe3, cp2)


# speedup vs baseline: 7.6988x; 1.1898x over previous
import functools

import jax
import jax.numpy as jnp
from jax import lax
from jax.experimental import pallas as pl
from jax.experimental.pallas import tpu as pltpu
from jax.experimental.pallas import tpu_sc as plsc

@functools.lru_cache(maxsize=None)
def _knn2_build(nq, nr, tq):
    grid = nq // tq

    def bf(x):
        return x.astype(jnp.bfloat16).astype(jnp.float32)

    def body(qx, qy, qz, rx, ry, rz, oi0, oi1, od0, od1):
        qxv, qyv, qzv = qx[...], qy[...], qz[...]
        rxv, ryv, rzv = rx[...], ry[...], rz[...]
        qq = (qxv * qxv + qzv * qzv) + qyv * qyv
        rr = (rxv * rxv + rzv * rzv) + ryv * ryv
        qr = (bf(qxv) * bf(rxv) + bf(qyv) * bf(ryv)) + bf(qzv) * bf(rzv)
        d = (qq + rr) - 2.0 * qr
        lane = lax.broadcasted_iota(jnp.int32, (tq, nr), 1)
        m1 = jnp.min(d, axis=1, keepdims=True)
        i1 = jnp.min(jnp.where(d == m1, lane, (2**30)), axis=1, keepdims=True)
        dm = jnp.where(lane == i1, jnp.inf, d)
        m2 = jnp.min(dm, axis=1, keepdims=True)
        i2 = jnp.min(jnp.where((d == m2) & (lane != i1), lane, (2**30)),
                     axis=1, keepdims=True)
        oi0[...] = i1
        oi1[...] = i2
        od0[...] = m1
        od1[...] = m2

    qspec = pl.BlockSpec((tq, 1), lambda i: (i, 0))
    rspec = pl.BlockSpec((1, nr), lambda i: (0, 0))
    ospec = pl.BlockSpec((tq, 1), lambda i: (i, 0))
    return pl.pallas_call(
        body,
        grid=(grid,),
        in_specs=[qspec] * 3 + [rspec] * 3,
        out_specs=[ospec] * 4,
        out_shape=[
            jax.ShapeDtypeStruct((nq, 1), jnp.int32),
            jax.ShapeDtypeStruct((nq, 1), jnp.int32),
            jax.ShapeDtypeStruct((nq, 1), jnp.float32),
            jax.ShapeDtypeStruct((nq, 1), jnp.float32),
        ],
    )


def _knn2(qc, rc, tq=128):
    nq = qc[0].shape[0]
    nr = rc[0].shape[1]
    return _knn2_build(nq, nr, tq)(*qc, *rc)


@functools.lru_cache(maxsize=None)
def _fps_build(n, m):
    rows = n // 128

    def body(xs, ys, zs, oidx, ox, oy, oz):
        xsv, ysv, zsv = xs[...], ys[...], zs[...]
        rowi = lax.broadcasted_iota(jnp.int32, (rows, 128), 0)
        coli = lax.broadcasted_iota(jnp.int32, (rows, 128), 1)
        flat = rowi * 128 + coli

        def red2(op, a):
            return op(op(a, axis=0, keepdims=True), axis=1, keepdims=True)

        def step(t, carry):
            last, dist = carry
            sel = flat == last
            xl = red2(jnp.max, jnp.where(sel, xsv, -1.0))
            yl = red2(jnp.max, jnp.where(sel, ysv, -1.0))
            zl = red2(jnp.max, jnp.where(sel, zsv, -1.0))
            oidx[pl.ds(t, 1), :] = last
            ox[pl.ds(t, 1), :] = xl
            oy[pl.ds(t, 1), :] = yl
            oz[pl.ds(t, 1), :] = zl
            dx = xsv - xl
            dy = ysv - yl
            dz = zsv - zl
            d = (dx * dx + dz * dz) + dy * dy
            nd = jnp.minimum(dist, d)
            mv = red2(jnp.max, nd)
            nxt = red2(jnp.min, jnp.where(nd == mv, flat, (2**30)))
            return (nxt, nd)

        init = (jnp.zeros((1, 1), jnp.int32),
                jnp.full((rows, 128), jnp.inf, jnp.float32))
        lax.fori_loop(0, m, step, init)

    return pl.pallas_call(
        body,
        out_shape=[
            jax.ShapeDtypeStruct((m, 1), jnp.int32),
            jax.ShapeDtypeStruct((m, 1), jnp.float32),
            jax.ShapeDtypeStruct((m, 1), jnp.float32),
            jax.ShapeDtypeStruct((m, 1), jnp.float32),
        ],
    )


def _fps(xc, yc, zc, m):
    n = xc.shape[0]
    xs = xc.reshape(n // 128, 128)
    ys = yc.reshape(n // 128, 128)
    zs = zc.reshape(n // 128, 128)
    return _fps_build(n, m)(xs, ys, zs)


def _dotbf(a, w):
    return jnp.dot(a.astype(jnp.bfloat16), w.astype(jnp.bfloat16),
                   preferred_element_type=jnp.float32)


@functools.lru_cache(maxsize=None)
def _prep_fe_build(n, c):
    npad = 125

    def body(f2, f1, wa, wb, b1, x2, y2, z2, og, oh):
        pad = jnp.broadcast_to(x2[...] * 0.0, (n, npad))
        og[...] = jnp.concatenate(
            [_dotbf(f2[...], wa[...]), x2[...], y2[...], z2[...], pad],
            axis=1)
        oh[...] = _dotbf(f1[...], wb[...]) + b1[...]

    return pl.pallas_call(
        body,
        out_shape=[
            jax.ShapeDtypeStruct((n, c + 3 + npad), jnp.float32),
            jax.ShapeDtypeStruct((n, c), jnp.float32),
        ],
    )


@functools.lru_cache(maxsize=None)
def _prep_sc_build(n, c):
    npad = 125

    def body(f, wa, xc, yc, zc, og):
        pad = jnp.broadcast_to(xc[...] * 0.0, (n, npad))
        og[...] = jnp.concatenate(
            [_dotbf(f[...], wa[...]), xc[...], yc[...], zc[...], pad],
            axis=1)

    return pl.pallas_call(
        body,
        out_shape=jax.ShapeDtypeStruct((n, c + 3 + npad), jnp.float32),
    )


@functools.lru_cache(maxsize=None)
def _gather_build(v, d, b):
    nw = 32
    b_per_w = b // nw
    chunk = min(b_per_w, 128)
    n_chunks = b_per_w // chunk
    mesh = plsc.VectorSubcoreMesh(core_axis_name="c", subcore_axis_name="s")

    @functools.partial(
        pl.kernel,
        mesh=mesh,
        out_type=jax.ShapeDtypeStruct((b, d), jnp.float32),
        scratch_types=[
            pltpu.VMEM((n_chunks, chunk), jnp.int32),
            pltpu.VMEM((chunk, d), jnp.float32),
            pltpu.SemaphoreType.DMA,
        ],
    )
    def k(idx_hbm, table_hbm, out_hbm, idx_v, rows_v, sem):
        wid = lax.axis_index("s") * 2 + lax.axis_index("c")
        pltpu.sync_copy(idx_hbm.at[pl.ds(wid * n_chunks, n_chunks)], idx_v)
        for j in range(n_chunks):
            pltpu.async_copy(table_hbm.at[idx_v.at[j]], rows_v, sem).wait()
            pltpu.sync_copy(
                rows_v, out_hbm.at[pl.ds(wid * b_per_w + j * chunk, chunk)])

    return k, chunk


def _gather(idx, table):
    v, d = table.shape
    b = idx.shape[0]
    k, chunk = _gather_build(v, d, b)
    return k(idx.reshape(b // chunk, chunk), table)


@functools.lru_cache(maxsize=None)
def _mlp_build(nq, c1, c2, c3, r2, h_rows):
    inv_cnt = 1.0 / (2.0 * nq)

    def bf(x):
        return x.astype(jnp.bfloat16).astype(jnp.float32)

    def bn(a0, a1, g, be):
        mu = (jnp.sum(a0, axis=0, keepdims=True)
              + jnp.sum(a1, axis=0, keepdims=True)) * inv_cnt
        d0 = a0 - mu
        d1 = a1 - mu
        var = (jnp.sum(d0 * d0, axis=0, keepdims=True)
               + jnp.sum(d1 * d1, axis=0, keepdims=True)) * inv_cnt
        s = jnp.sqrt(var + 1e-5)
        return d0 / s * g + be, d1 / s * g + be

    def body(z0, z1, h, pc, w0, w1, w2p, w2, b2, w3, b3,
             g1, be1, g2, be2, g3, be3, out):
        w0v, w1v, w2v = bf(w0[...]), bf(w1[...]), bf(w2p[...])
        pcv = pc[...]
        qxv = pcv[:, 0:1]
        qyv = pcv[:, 1:2]
        qzv = pcv[:, 2:3]
        dv0 = pcv[:, 3:4]
        dv1 = pcv[:, 4:5]
        hv = h[...]

        def layer1(z):
            zv = z[...]
            rx = bf(zv[:, c1:c1 + 1] - qxv)
            ry = bf(zv[:, c1 + 1:c1 + 2] - qyv)
            rz = bf(zv[:, c1 + 2:c1 + 3] - qzv)
            return (zv[:, 0:c1] + hv) + ((rx * w0v + ry * w1v) + rz * w2v)

        a0 = layer1(z0)
        a1 = layer1(z1)
        a0, a1 = bn(a0, a1, g1[...], be1[...])
        a0 = jnp.maximum(a0, 0.0)
        a1 = jnp.maximum(a1, 0.0)
        t0 = _dotbf(a0, w2[...]) + b2[...]
        t1 = _dotbf(a1, w2[...]) + b2[...]
        a0, a1 = bn(t0, t1, g2[...], be2[...])
        a0 = jnp.maximum(a0, 0.0)
        a1 = jnp.maximum(a1, 0.0)
        t0 = _dotbf(a0, w3[...]) + b3[...]
        t1 = _dotbf(a1, w3[...]) + b3[...]
        a0, a1 = bn(t0, t1, g3[...], be3[...])
        a0 = jnp.maximum(a0, 0.0)
        a1 = jnp.maximum(a1, 0.0)
        mask0 = dv0 <= r2
        mask1 = dv1 <= r2
        mx = jnp.maximum(jnp.where(mask0, a0, -1e30),
                         jnp.where(mask1, a1, -1e30))
        out[...] = jnp.where(mask0 | mask1, mx, 0.0)

    return pl.pallas_call(
        body,
        out_shape=jax.ShapeDtypeStruct((nq, c3), jnp.float32),
    )


def _mlp(z0, z1, h, qc, wc_rows, dv, lyr2, lyr3, gbe1, r2):
    nq = z0.shape[0]
    c1 = lyr2[0].shape[0]
    w2, b2, g2, be2 = lyr2
    w3, b3, g3, be3 = lyr3
    g1, be1 = gbe1
    pc = jnp.concatenate(list(qc) + [dv[0], dv[1]], axis=1)
    return _mlp_build(nq, c1, w2.shape[1], w3.shape[1], r2, h.shape[0])(
        z0, z1, h, pc, *wc_rows, w2, _r(b2), w3, _r(b3),
        _r(g1), _r(be1), _r(g2), _r(be2), _r(g3), _r(be3))


def _cols(p):
    return p[:, 0:1], p[:, 1:2], p[:, 2:3]


def _r(x):
    return x.reshape(1, -1)


def kernel(x1_features, x1_pos, x1_batch, x2_features, x2_pos, x2_batch,
           params):
    n = x1_features.shape[0]
    p1 = x1_pos
    x1c, y1c, z1c = _cols(p1)
    x2c, y2c, z2c = _cols(x2_pos)

    (w1, b1, g1, be1), l2, l3 = params['fe']
    wa, wb, wc = w1[0:128], w1[128:256], w1[256:259]
    wc_rows = (_r(wc[0]), _r(wc[1]), _r(wc[2]))
    i0, i1, d0, d1 = _knn2((x1c, y1c, z1c), (_r(x2c), _r(y2c), _r(z2c)))
    g_tab, h_q = _prep_fe_build(n, 128)(
        x2_features, x1_features, wa, wb, _r(b1), x2c, y2c, z2c)
    z = _gather(jnp.concatenate([i0.reshape(-1), i1.reshape(-1)]), g_tab)
    fe1 = _mlp(z[:n], z[n:], h_q, (x1c, y1c, z1c), wc_rows, (d0, d1),
               l2, l3, (g1, be1), 25.0)

    (w1, b1, g1, be1), l2, l3 = params['sc1']
    wa, wc = w1[0:128], w1[128:131]
    wc_rows = (_r(wc[0]), _r(wc[1]), _r(wc[2]))
    m1 = 2048
    _, cx, cy, cz = _fps(x1c, y1c, z1c, m1)
    cp = jnp.concatenate([cx, cy, cz], axis=1)
    i0, i1, d0, d1 = _knn2((cx, cy, cz), (_r(x1c), _r(y1c), _r(z1c)))
    g_tab = _prep_sc_build(n, 128)(fe1, wa, x1c, y1c, z1c)
    z = _gather(jnp.concatenate([i0.reshape(-1), i1.reshape(-1)]), g_tab)
    fe2 = _mlp(z[:m1], z[m1:], _r(b1), (cx, cy, cz), wc_rows, (d0, d1),
               l2, l3, (g1, be1), 4.0)

    (w1, b1, g1, be1), l2, l3 = params['sc2']
    wa, wc = w1[0:256], w1[256:259]
    wc_rows = (_r(wc[0]), _r(wc[1]), _r(wc[2]))
    m2 = 512
    _, qx, qy, qz = _fps(cx, cy, cz, m2)
    cp2 = jnp.concatenate([qx, qy, qz], axis=1)
    i0, i1, d0, d1 = _knn2((qx, qy, qz), (_r(cx), _r(cy), _r(cz)))
    g_tab = _prep_sc_build(m1, 256)(fe2, wa, cx, cy, cz)
    z = _gather(jnp.concatenate([i0.reshape(-1), i1.reshape(-1)]), g_tab)
    fe3 = _mlp(z[:m2], z[m2:], _r(b1), (qx, qy, qz), wc_rows, (d0, d1),
               l2, l3, (g1, be1), 16.0)

    return (fe1, p1, fe2, cp, fe3, cp2)
